# Initial kernel scaffold; baseline (speedup 1.0000x reference)
import jax
import jax.numpy as jnp
from jax.experimental import pallas as pl


def kernel(pred_logits, gt):
    raise NotImplementedError("write your pallas kernel here")


# speedup vs baseline: 104.4207x; 104.4207x over previous
import jax
import jax.numpy as jnp
from jax import lax
from jax.experimental import pallas as pl
from jax.experimental.pallas import tpu as pltpu

_NEG_RATIO = 3.0
_EPS = 1e-6
_N = 8 * 512 * 512
_ROWS = 4096
_COLS = 512
_BLK = 128
_GRID = _ROWS // _BLK


def _bce(x, z):
    return jnp.maximum(x, 0.0) - x * z + jnp.log1p(jnp.exp(-jnp.abs(x)))


def _reduce_body(pred_ref, gt_ref, out_ref, acc_ref):
    i = pl.program_id(0)

    @pl.when(i == 0)
    def _():
        acc_ref[0] = 0.0
        acc_ref[1] = 0.0
        acc_ref[2] = 0.0

    x = pred_ref[...]
    z = gt_ref[...]
    loss = _bce(x, z)
    acc_ref[0] += jnp.sum(loss)
    acc_ref[1] += jnp.sum(loss * z)
    acc_ref[2] += jnp.sum(z)

    @pl.when(i == pl.num_programs(0) - 1)
    def _():
        out_ref[0] = acc_ref[0]
        out_ref[1] = acc_ref[1]
        out_ref[2] = acc_ref[2]


_reduce = pl.pallas_call(
    _reduce_body,
    grid=(_GRID,),
    in_specs=[
        pl.BlockSpec((_BLK, _COLS), lambda i: (i, 0)),
        pl.BlockSpec((_BLK, _COLS), lambda i: (i, 0)),
    ],
    out_specs=pl.BlockSpec(memory_space=pltpu.SMEM),
    out_shape=jax.ShapeDtypeStruct((3,), jnp.float32),
    scratch_shapes=[pltpu.SMEM((3,), jnp.float32)],
)


def _topk_body(k_ref, pred_ref, gt_ref, out_ref, nl_ref):
    x = pred_ref[...]
    z = gt_ref[...]
    loss = _bce(x, z)
    nl_ref[...] = jnp.where((1.0 - z) > 0, loss, 0.0)
    k_f = k_ref[0].astype(jnp.float32)

    def body(i, cur):
        bits = lax.bitcast_convert_type(nl_ref[...], jnp.int32)
        t = cur + lax.shift_left(jnp.int32(1), 30 - i)
        cnt = jnp.sum((bits >= t).astype(jnp.float32))
        return jnp.where(cnt >= k_f, t, cur)

    cur = lax.fori_loop(0, 31, body, jnp.int32(0))
    nl = nl_ref[...]
    bits = lax.bitcast_convert_type(nl, jnp.int32)
    kth_val = jnp.max(jnp.where(bits == cur, nl, 0.0))
    gt_mask = bits > cur
    sum_gt = jnp.sum(jnp.where(gt_mask, nl, 0.0))
    cnt_gt = jnp.sum(gt_mask.astype(jnp.float32))
    res = sum_gt + (k_f - cnt_gt) * kth_val
    out_ref[0] = jnp.where(k_f > 0, res, 0.0)


_topk = pl.pallas_call(
    _topk_body,
    in_specs=[
        pl.BlockSpec(memory_space=pltpu.SMEM),
        pl.BlockSpec(memory_space=pltpu.VMEM),
        pl.BlockSpec(memory_space=pltpu.VMEM),
    ],
    out_specs=pl.BlockSpec(memory_space=pltpu.SMEM),
    out_shape=jax.ShapeDtypeStruct((1,), jnp.float32),
    scratch_shapes=[pltpu.VMEM((_ROWS, _COLS), jnp.float32)],
)


def kernel(pred_logits, gt):
    p2 = pred_logits.reshape(_ROWS, _COLS)
    z2 = gt.reshape(_ROWS, _COLS)
    sums = _reduce(p2, z2)
    tot, pos_sum, pos_f = sums[0], sums[1], sums[2]
    pos_i = pos_f.astype(jnp.int32)
    neg_i = jnp.int32(_N) - pos_i
    cap = (pos_f * _NEG_RATIO).astype(jnp.int32)
    k = jnp.minimum(neg_i, cap)
    denom = pos_f + k.astype(jnp.float32) + _EPS

    def easy(_):
        return (pos_sum + (tot - pos_sum)) / denom

    def hard(_):
        tk = _topk(k.reshape(1), p2, z2)[0]
        return (pos_sum + tk) / denom

    return lax.cond(k >= neg_i, easy, hard, None)
